# Initial kernel scaffold; baseline (speedup 1.0000x reference)
import jax
import jax.numpy as jnp
from jax import lax
from jax.experimental import pallas as pl
from jax.experimental.pallas import tpu as pltpu
from jax.experimental.pallas import tpu_sc as plsc

N = 10000
E = 320000
NC, NS = 2, 16
NW = NC * NS
RPT = 640
NPAD = NS * RPT
EB = 512
NBLK = E // EB
TPW = (NBLK + NW - 1) // NW

_mesh = plsc.VectorSubcoreMesh(core_axis_name="c", subcore_axis_name="s")


def _sc1_body(col3, out, zbuf, ones_buf, cbuf, cv2, acc):
    c = lax.axis_index("c")
    s = lax.axis_index("s")
    w = c * NS + s
    z16 = jnp.zeros((16,), jnp.float32)
    o16 = jnp.ones((16,), jnp.float32)

    def fz(i, _):
        zbuf[i] = z16
        return 0
    lax.fori_loop(0, RPT, fz, 0)

    def fo(i, _):
        ones_buf[i] = o16
        return 0
    lax.fori_loop(0, EB, fo, 0)

    pltpu.sync_copy(zbuf, acc.at[pl.ds(s * RPT, RPT)])
    plsc.subcore_barrier()

    def blk_body(t, _):
        blk = w + NW * t

        @pl.when(blk < NBLK)
        def _():
            pltpu.sync_copy(col3.at[blk], cv2)
            for j in range(EB // 128):
                pltpu.sync_copy(ones_buf.at[pl.ds(j * 128, 128)],
                                acc.at[cv2.at[j]], add=True)
        return 0
    lax.fori_loop(0, TPW, blk_body, 0)
    plsc.subcore_barrier()

    pltpu.sync_copy(acc.at[pl.ds(s * RPT, RPT)], zbuf)
    ii = lax.iota(jnp.int32, 16)
    z16i = jnp.zeros((16,), jnp.int32)

    def fc(k, _):
        g = plsc.load_gather(zbuf, [ii + k * 16, z16i])
        cbuf[pl.ds(k * 16, 16)] = g
        return 0
    lax.fori_loop(0, RPT // 16, fc, 0)
    pltpu.sync_copy(cbuf, out.at[c, pl.ds(s * RPT, RPT)])


_sc1 = pl.kernel(
    _sc1_body,
    out_type=jax.ShapeDtypeStruct((NC, NPAD), jnp.float32),
    mesh=_mesh,
    scratch_types=[
        pltpu.VMEM((RPT, 16), jnp.float32),
        pltpu.VMEM((EB, 16), jnp.float32),
        pltpu.VMEM((RPT,), jnp.float32),
        pltpu.VMEM((EB // 128, 128), jnp.int32),
        pltpu.VMEM_SHARED((NPAD, 16), jnp.float32),
    ],
)


def _sc2_body(row3, col3, cp, a_t, b_t, dgc, out,
              p0, p1, rv2, cv2, abuf, bbuf, cbuf, obuf, sa, sb, sc, acc):
    c = lax.axis_index("c")
    s = lax.axis_index("s")
    w = c * NS + s
    z16 = jnp.zeros((16,), jnp.float32)
    o16 = jnp.ones((16,), jnp.float32)
    ii = lax.iota(jnp.int32, 16)

    def fz(i, _):
        obuf[i, pl.ds(0, 16)] = z16
        obuf[i, pl.ds(16, 16)] = z16
        return 0
    lax.fori_loop(0, EB, fz, 0)
    pltpu.sync_copy(obuf, acc.at[pl.ds(s * RPT, EB)])
    pltpu.sync_copy(obuf.at[pl.ds(0, RPT - EB)],
                    acc.at[pl.ds(s * RPT + EB, RPT - EB)])
    c17 = jnp.full((16,), 17, jnp.int32)

    def f17(g, _):
        plsc.store_scatter(obuf, [g * 16 + ii, c17], o16)
        return 0
    lax.fori_loop(0, EB // 16, f17, 0)

    pltpu.sync_copy(dgc.at[0], p0)
    pltpu.sync_copy(dgc.at[1], p1)
    plsc.subcore_barrier()

    c16 = jnp.full((16,), 16, jnp.int32)

    def blk_body(t, _):
        blk = w + NW * t

        @pl.when(blk < NBLK)
        def _():
            pltpu.sync_copy(row3.at[blk], rv2)
            pltpu.sync_copy(col3.at[blk], cv2)
            cps = []
            for j in range(EB // 128):
                cps.append(pltpu.async_copy(
                    a_t.at[rv2.at[j]], abuf.at[pl.ds(j * 128, 128)], sa))
                cps.append(pltpu.async_copy(
                    b_t.at[cv2.at[j]], bbuf.at[pl.ds(j * 128, 128)], sb))
            cps.append(pltpu.async_copy(
                cp.at[pl.ds(blk * EB, EB)], cbuf, sc))
            for d in cps:
                d.wait()

            def grp(g, _):
                j = g // 8
                o = (g % 8) * 16
                cvv = cv2[j, pl.ds(o, 16)]
                d = plsc.load_gather(p0, [cvv]) + plsc.load_gather(p1, [cvv])
                wv = 1.0 / jnp.maximum(d, 1.0)
                plsc.store_scatter(obuf, [g * 16 + ii, c16], wv)

                def edge(e, _):
                    r = g * 16 + e
                    h = jnp.maximum(abuf[r] + bbuf[r] + cbuf[r], 0.0)
                    obuf[r, pl.ds(0, 16)] = h
                    return 0
                lax.fori_loop(0, 16, edge, 0)
                return 0
            lax.fori_loop(0, EB // 16, grp, 0)

            for j in range(EB // 128):
                pltpu.sync_copy(obuf.at[pl.ds(j * 128, 128)],
                                acc.at[rv2.at[j]], add=True)
        return 0
    lax.fori_loop(0, TPW, blk_body, 0)
    plsc.subcore_barrier()
    pltpu.sync_copy(acc.at[pl.ds(s * RPT, RPT)],
                    out.at[c, pl.ds(s * RPT, RPT)])


_sc2 = pl.kernel(
    _sc2_body,
    out_type=jax.ShapeDtypeStruct((NC, NPAD, 32), jnp.float32),
    mesh=_mesh,
    scratch_types=[
        pltpu.VMEM((NPAD,), jnp.float32),
        pltpu.VMEM((NPAD,), jnp.float32),
        pltpu.VMEM((EB // 128, 128), jnp.int32),
        pltpu.VMEM((EB // 128, 128), jnp.int32),
        pltpu.VMEM((EB, 16), jnp.float32),
        pltpu.VMEM((EB, 16), jnp.float32),
        pltpu.VMEM((EB, 16), jnp.float32),
        pltpu.VMEM((EB, 32), jnp.float32),
        pltpu.SemaphoreType.DMA,
        pltpu.SemaphoreType.DMA,
        pltpu.SemaphoreType.DMA,
        pltpu.VMEM_SHARED((NPAD, 32), jnp.float32),
    ],
)


BN = 2000
BE = 16000


def _tca_body(x_ref, wne, bne, ga, wge, bge, wr, wc, wg, bea,
              h_ref, a_ref, b_ref, hg_ref, kv_ref):
    i = pl.program_id(0)
    h = jnp.dot(x_ref[...], wne[...], preferred_element_type=jnp.float32) + bne[...]
    h_ref[...] = h
    a_ref[...] = jnp.dot(h, wr[...], preferred_element_type=jnp.float32)
    b_ref[...] = jnp.dot(h, wc[...], preferred_element_type=jnp.float32)

    @pl.when(i == 0)
    def _():
        hg = jnp.dot(ga[...], wge[...], preferred_element_type=jnp.float32) + bge[...]
        hg_ref[...] = hg
        kv_ref[...] = jnp.dot(hg, wg[...], preferred_element_type=jnp.float32) + bea[...]


def _tcb_body(ea_ref, we, kv, c_ref):
    c_ref[...] = (jnp.dot(ea_ref[...], we[...], preferred_element_type=jnp.float32)
                  + kv[...])


def _tcf_body(h_ref, acc_ref, wna1, wna2, hg, wna3, bna, wl, wr2, bsc,
              out_ref, s1_ref, s2_ref):
    i = pl.program_id(0)

    @pl.when(i == 0)
    def _():
        s1_ref[...] = jnp.zeros_like(s1_ref)
        s2_ref[...] = jnp.zeros_like(s2_ref)

    a = acc_ref[...]
    m_sum = a[0, :, :16] + a[1, :, :16]
    cntw = a[0, :, 16] + a[1, :, 16]
    deg = a[0, :, 17] + a[1, :, 17]
    m_n = m_sum / jnp.maximum(deg, 1.0)[:, None]
    h = h_ref[...]
    cna = jnp.dot(hg[...], wna3[...], preferred_element_type=jnp.float32) + bna[...]
    pre = (jnp.dot(h, wna1[...], preferred_element_type=jnp.float32)
           + jnp.dot(m_n, wna2[...], preferred_element_type=jnp.float32) + cna)
    h2 = jnp.maximum(pre, 0.0)
    s1_ref[...] += jnp.dot(cntw[None, :], h2, preferred_element_type=jnp.float32)
    s2_ref[...] += jnp.sum(h2, axis=0, keepdims=True)

    @pl.when(i == pl.num_programs(0) - 1)
    def _():
        pooled = ((jnp.dot(s1_ref[...], wl[...], preferred_element_type=jnp.float32)
                   + jnp.dot(s2_ref[...], wr2[...], preferred_element_type=jnp.float32))
                  / N + bsc[...])
        m = jnp.max(pooled)
        lse = m + jnp.log(jnp.sum(jnp.exp(pooled - m)))
        out_ref[...] = pooled - lse


def kernel(x, edge_index, edge_attr, graph_attr, batch,
           W_ne, b_ne, W_ge, b_ge, W_ea, b_ea, W_na, b_na,
           W_ga, b_ga, W_l, W_r, b_sc):
    del batch, W_ga, b_ga
    row3 = edge_index[0].reshape(NBLK, EB // 128, 128)
    col3 = edge_index[1].reshape(NBLK, EB // 128, 128)

    h_i, a_t, b_t, hg, kv = pl.pallas_call(
        _tca_body,
        grid=(N // BN,),
        in_specs=[
            pl.BlockSpec((BN, 128), lambda i: (i, 0)),
            pl.BlockSpec((128, 64), lambda i: (0, 0)),
            pl.BlockSpec((1, 64), lambda i: (0, 0)),
            pl.BlockSpec((1, 16), lambda i: (0, 0)),
            pl.BlockSpec((16, 16), lambda i: (0, 0)),
            pl.BlockSpec((1, 16), lambda i: (0, 0)),
            pl.BlockSpec((64, 16), lambda i: (0, 0)),
            pl.BlockSpec((64, 16), lambda i: (0, 0)),
            pl.BlockSpec((16, 16), lambda i: (0, 0)),
            pl.BlockSpec((1, 16), lambda i: (0, 0)),
        ],
        out_specs=[
            pl.BlockSpec((BN, 64), lambda i: (i, 0)),
            pl.BlockSpec((BN, 16), lambda i: (i, 0)),
            pl.BlockSpec((BN, 16), lambda i: (i, 0)),
            pl.BlockSpec((1, 16), lambda i: (0, 0)),
            pl.BlockSpec((1, 16), lambda i: (0, 0)),
        ],
        out_shape=[
            jax.ShapeDtypeStruct((N, 64), jnp.float32),
            jax.ShapeDtypeStruct((N, 16), jnp.float32),
            jax.ShapeDtypeStruct((N, 16), jnp.float32),
            jax.ShapeDtypeStruct((1, 16), jnp.float32),
            jax.ShapeDtypeStruct((1, 16), jnp.float32),
        ],
    )(x, W_ne, b_ne.reshape(1, 64), graph_attr, W_ge, b_ge.reshape(1, 16),
      W_ea[:64], W_ea[64:128], W_ea[144:160], b_ea.reshape(1, 16))

    cp = pl.pallas_call(
        _tcb_body,
        grid=(E // BE,),
        in_specs=[
            pl.BlockSpec((BE, 16), lambda i: (i, 0)),
            pl.BlockSpec((16, 16), lambda i: (0, 0)),
            pl.BlockSpec((1, 16), lambda i: (0, 0)),
        ],
        out_specs=pl.BlockSpec((BE, 16), lambda i: (i, 0)),
        out_shape=jax.ShapeDtypeStruct((E, 16), jnp.float32),
    )(edge_attr, W_ea[128:144], kv)

    dgc = _sc1(col3)
    acc2 = _sc2(row3, col3, cp, a_t, b_t, dgc)

    out = pl.pallas_call(
        _tcf_body,
        grid=(N // BN,),
        in_specs=[
            pl.BlockSpec((BN, 64), lambda i: (i, 0)),
            pl.BlockSpec((2, BN, 32), lambda i: (0, i, 0)),
            pl.BlockSpec((64, 64), lambda i: (0, 0)),
            pl.BlockSpec((16, 64), lambda i: (0, 0)),
            pl.BlockSpec((1, 16), lambda i: (0, 0)),
            pl.BlockSpec((16, 64), lambda i: (0, 0)),
            pl.BlockSpec((1, 64), lambda i: (0, 0)),
            pl.BlockSpec((64, 64), lambda i: (0, 0)),
            pl.BlockSpec((64, 64), lambda i: (0, 0)),
            pl.BlockSpec((1, 64), lambda i: (0, 0)),
        ],
        out_specs=pl.BlockSpec((1, 64), lambda i: (0, 0)),
        out_shape=jax.ShapeDtypeStruct((1, 64), jnp.float32),
        scratch_shapes=[
            pltpu.VMEM((1, 64), jnp.float32),
            pltpu.VMEM((1, 64), jnp.float32),
        ],
    )(h_i, acc2[:, :N, :], W_na[:64], W_na[64:80], hg, W_na[80:96],
      b_na.reshape(1, 64), W_l, W_r, b_sc.reshape(1, 64))

    return out


# speedup vs baseline: 7.7770x; 7.7770x over previous
import jax
import jax.numpy as jnp
from jax import lax
from jax.experimental import pallas as pl
from jax.experimental.pallas import tpu as pltpu
from jax.experimental.pallas import tpu_sc as plsc

N = 10000
E = 320000
NC, NS = 2, 16
NW = NC * NS
RPT = 640
NPAD = NS * RPT
EB = 512
NBLK = E // EB
TPW = (NBLK + NW - 1) // NW

_mesh = plsc.VectorSubcoreMesh(core_axis_name="c", subcore_axis_name="s")
_sc_params = pltpu.CompilerParams(use_tc_tiling_on_sc=False,
                                 needs_layout_passes=False)


def _sc1_body(col3, out, zbuf, ones_buf, cv2, acc):
    c = lax.axis_index("c")
    s = lax.axis_index("s")
    w = c * NS + s
    z16 = jnp.zeros((16,), jnp.float32)
    o16 = jnp.ones((16,), jnp.float32)

    def fz(i, _):
        zbuf[i] = z16
        return 0
    lax.fori_loop(0, RPT, fz, 0)

    def fo(i, _):
        ones_buf[i] = o16
        return 0
    lax.fori_loop(0, EB, fo, 0)

    pltpu.sync_copy(zbuf, acc.at[pl.ds(s * RPT, RPT)])
    plsc.subcore_barrier()

    def blk_body(t, _):
        blk = w + NW * t

        @pl.when(blk < NBLK)
        def _():
            pltpu.sync_copy(col3.at[blk], cv2)
            for j in range(EB // 128):
                pltpu.sync_copy(ones_buf.at[pl.ds(j * 128, 128)],
                                acc.at[cv2.at[j]], add=True)
        return 0
    lax.fori_loop(0, TPW, blk_body, 0)
    plsc.subcore_barrier()
    pltpu.sync_copy(acc.at[pl.ds(s * RPT, RPT)],
                    out.at[c, pl.ds(s * RPT, RPT)])


_sc1 = pl.kernel(
    _sc1_body,
    out_type=jax.ShapeDtypeStruct((NC, NPAD, 16), jnp.float32),
    mesh=_mesh,
    compiler_params=_sc_params,
    scratch_types=[
        pltpu.VMEM((RPT, 16), jnp.float32),
        pltpu.VMEM((EB, 16), jnp.float32),
        pltpu.VMEM((EB // 128, 128), jnp.int32),
        pltpu.VMEM_SHARED((NPAD, 16), jnp.float32),
    ],
)


def _tcr_body(dgc_ref, rp_ref):
    a = dgc_ref[...]
    degc = a[0, :, 0] + a[1, :, 0]
    rp_ref[...] = (1.0 / jnp.maximum(degc, 1.0))[None, :]


def _sc2_body(row3, col3, cp, a_t, b_t, rp_in, out,
              rp, rv2, cv2, abuf, bbuf, cbuf, obuf, sa, sb, sc, acc):
    c = lax.axis_index("c")
    s = lax.axis_index("s")
    w = c * NS + s
    z16 = jnp.zeros((16,), jnp.float32)
    o16 = jnp.ones((16,), jnp.float32)
    ii = lax.iota(jnp.int32, 16)

    def fz(i, _):
        obuf[i, pl.ds(0, 16)] = z16
        obuf[i, pl.ds(16, 16)] = z16
        return 0
    lax.fori_loop(0, EB, fz, 0)
    pltpu.sync_copy(obuf, acc.at[pl.ds(s * RPT, EB)])
    pltpu.sync_copy(obuf.at[pl.ds(0, RPT - EB)],
                    acc.at[pl.ds(s * RPT + EB, RPT - EB)])

    pltpu.sync_copy(rp_in.at[0], rp)
    plsc.subcore_barrier()

    c_ii1 = jnp.where(ii == 1, 1.0, 0.0).astype(jnp.float32)

    def blk_body(t, _):
        blk = w + NW * t

        @pl.when(blk < NBLK)
        def _():
            pltpu.sync_copy(row3.at[blk], rv2)
            pltpu.sync_copy(col3.at[blk], cv2)
            cps = []
            for j in range(EB // 128):
                cps.append(pltpu.async_copy(
                    a_t.at[rv2.at[j]], abuf.at[pl.ds(j * 128, 128)], sa))
                cps.append(pltpu.async_copy(
                    b_t.at[cv2.at[j]], bbuf.at[pl.ds(j * 128, 128)], sb))
            cps.append(pltpu.async_copy(
                cp.at[pl.ds(blk * EB, EB)], cbuf, sc))
            for d in cps:
                d.wait()

            def grp(g, _):
                j = g // 8
                o = (g % 8) * 16
                cvv = cv2[j, pl.ds(o, 16)]
                wv = plsc.load_gather(rp, [cvv])
                for e in range(16):
                    r = g * 16 + e
                    h = jnp.maximum(abuf[r] + bbuf[r] + cbuf[r], 0.0)
                    upper = jnp.where(ii == 0, wv[e], c_ii1)
                    obuf[r, pl.ds(0, 16)] = h
                    obuf[r, pl.ds(16, 16)] = upper
                return 0
            lax.fori_loop(0, EB // 16, grp, 0)

            for j in range(EB // 128):
                pltpu.sync_copy(obuf.at[pl.ds(j * 128, 128)],
                                acc.at[rv2.at[j]], add=True)
        return 0
    lax.fori_loop(0, TPW, blk_body, 0)
    plsc.subcore_barrier()
    pltpu.sync_copy(acc.at[pl.ds(s * RPT, RPT)],
                    out.at[c, pl.ds(s * RPT, RPT)])


_sc2 = pl.kernel(
    _sc2_body,
    out_type=jax.ShapeDtypeStruct((NC, NPAD, 32), jnp.float32),
    mesh=_mesh,
    compiler_params=_sc_params,
    scratch_types=[
        pltpu.VMEM((NPAD,), jnp.float32),
        pltpu.VMEM((EB // 128, 128), jnp.int32),
        pltpu.VMEM((EB // 128, 128), jnp.int32),
        pltpu.VMEM((EB, 16), jnp.float32),
        pltpu.VMEM((EB, 16), jnp.float32),
        pltpu.VMEM((EB, 16), jnp.float32),
        pltpu.VMEM((EB, 32), jnp.float32),
        pltpu.SemaphoreType.DMA,
        pltpu.SemaphoreType.DMA,
        pltpu.SemaphoreType.DMA,
        pltpu.VMEM_SHARED((NPAD, 32), jnp.float32),
    ],
)


BN = 2000
BE = 16000


def _tca_body(x_ref, wne, bne, ga, wge, bge, wr, wc, wg, bea,
              h_ref, a_ref, b_ref, hg_ref, kv_ref):
    i = pl.program_id(0)
    h = jnp.dot(x_ref[...], wne[...], preferred_element_type=jnp.float32) + bne[...]
    h_ref[...] = h
    a_ref[...] = jnp.dot(h, wr[...], preferred_element_type=jnp.float32)
    b_ref[...] = jnp.dot(h, wc[...], preferred_element_type=jnp.float32)

    @pl.when(i == 0)
    def _():
        hg = jnp.dot(ga[...], wge[...], preferred_element_type=jnp.float32) + bge[...]
        hg_ref[...] = hg
        kv_ref[...] = jnp.dot(hg, wg[...], preferred_element_type=jnp.float32) + bea[...]


def _tcb_body(ea_ref, we, kv, c_ref):
    c_ref[...] = (jnp.dot(ea_ref[...], we[...], preferred_element_type=jnp.float32)
                  + kv[...])


def _tcf_body(h_ref, acc_ref, wna1, wna2, hg, wna3, bna, wl, wr2, bsc,
              out_ref, s1_ref, s2_ref):
    i = pl.program_id(0)

    @pl.when(i == 0)
    def _():
        s1_ref[...] = jnp.zeros_like(s1_ref)
        s2_ref[...] = jnp.zeros_like(s2_ref)

    a = acc_ref[...]
    m_sum = a[0, :, :16] + a[1, :, :16]
    cntw = a[0, :, 16] + a[1, :, 16]
    deg = a[0, :, 17] + a[1, :, 17]
    m_n = m_sum / jnp.maximum(deg, 1.0)[:, None]
    h = h_ref[...]
    cna = jnp.dot(hg[...], wna3[...], preferred_element_type=jnp.float32) + bna[...]
    pre = (jnp.dot(h, wna1[...], preferred_element_type=jnp.float32)
           + jnp.dot(m_n, wna2[...], preferred_element_type=jnp.float32) + cna)
    h2 = jnp.maximum(pre, 0.0)
    s1_ref[...] += jnp.dot(cntw[None, :], h2, preferred_element_type=jnp.float32)
    s2_ref[...] += jnp.sum(h2, axis=0, keepdims=True)

    @pl.when(i == pl.num_programs(0) - 1)
    def _():
        pooled = ((jnp.dot(s1_ref[...], wl[...], preferred_element_type=jnp.float32)
                   + jnp.dot(s2_ref[...], wr2[...], preferred_element_type=jnp.float32))
                  / N + bsc[...])
        m = jnp.max(pooled)
        lse = m + jnp.log(jnp.sum(jnp.exp(pooled - m)))
        out_ref[...] = pooled - lse


def kernel(x, edge_index, edge_attr, graph_attr, batch,
           W_ne, b_ne, W_ge, b_ge, W_ea, b_ea, W_na, b_na,
           W_ga, b_ga, W_l, W_r, b_sc):
    del batch, W_ga, b_ga
    row3 = edge_index[0].reshape(NBLK, EB // 128, 128)
    col3 = edge_index[1].reshape(NBLK, EB // 128, 128)

    h_i, a_t, b_t, hg, kv = pl.pallas_call(
        _tca_body,
        grid=(N // BN,),
        in_specs=[
            pl.BlockSpec((BN, 128), lambda i: (i, 0)),
            pl.BlockSpec((128, 64), lambda i: (0, 0)),
            pl.BlockSpec((1, 64), lambda i: (0, 0)),
            pl.BlockSpec((1, 16), lambda i: (0, 0)),
            pl.BlockSpec((16, 16), lambda i: (0, 0)),
            pl.BlockSpec((1, 16), lambda i: (0, 0)),
            pl.BlockSpec((64, 16), lambda i: (0, 0)),
            pl.BlockSpec((64, 16), lambda i: (0, 0)),
            pl.BlockSpec((16, 16), lambda i: (0, 0)),
            pl.BlockSpec((1, 16), lambda i: (0, 0)),
        ],
        out_specs=[
            pl.BlockSpec((BN, 64), lambda i: (i, 0)),
            pl.BlockSpec((BN, 16), lambda i: (i, 0)),
            pl.BlockSpec((BN, 16), lambda i: (i, 0)),
            pl.BlockSpec((1, 16), lambda i: (0, 0)),
            pl.BlockSpec((1, 16), lambda i: (0, 0)),
        ],
        out_shape=[
            jax.ShapeDtypeStruct((N, 64), jnp.float32),
            jax.ShapeDtypeStruct((N, 16), jnp.float32),
            jax.ShapeDtypeStruct((N, 16), jnp.float32),
            jax.ShapeDtypeStruct((1, 16), jnp.float32),
            jax.ShapeDtypeStruct((1, 16), jnp.float32),
        ],
    )(x, W_ne, b_ne.reshape(1, 64), graph_attr, W_ge, b_ge.reshape(1, 16),
      W_ea[:64], W_ea[64:128], W_ea[144:160], b_ea.reshape(1, 16))

    cp = pl.pallas_call(
        _tcb_body,
        grid=(E // BE,),
        in_specs=[
            pl.BlockSpec((BE, 16), lambda i: (i, 0)),
            pl.BlockSpec((16, 16), lambda i: (0, 0)),
            pl.BlockSpec((1, 16), lambda i: (0, 0)),
        ],
        out_specs=pl.BlockSpec((BE, 16), lambda i: (i, 0)),
        out_shape=jax.ShapeDtypeStruct((E, 16), jnp.float32),
    )(edge_attr, W_ea[128:144], kv)

    dgc = _sc1(col3)
    rp1 = pl.pallas_call(
        _tcr_body,
        out_shape=jax.ShapeDtypeStruct((1, NPAD), jnp.float32),
    )(dgc)
    acc2 = _sc2(row3, col3, cp, a_t, b_t, rp1)

    out = pl.pallas_call(
        _tcf_body,
        grid=(N // BN,),
        in_specs=[
            pl.BlockSpec((BN, 64), lambda i: (i, 0)),
            pl.BlockSpec((2, BN, 32), lambda i: (0, i, 0)),
            pl.BlockSpec((64, 64), lambda i: (0, 0)),
            pl.BlockSpec((16, 64), lambda i: (0, 0)),
            pl.BlockSpec((1, 16), lambda i: (0, 0)),
            pl.BlockSpec((16, 64), lambda i: (0, 0)),
            pl.BlockSpec((1, 64), lambda i: (0, 0)),
            pl.BlockSpec((64, 64), lambda i: (0, 0)),
            pl.BlockSpec((64, 64), lambda i: (0, 0)),
            pl.BlockSpec((1, 64), lambda i: (0, 0)),
        ],
        out_specs=pl.BlockSpec((1, 64), lambda i: (0, 0)),
        out_shape=jax.ShapeDtypeStruct((1, 64), jnp.float32),
        scratch_shapes=[
            pltpu.VMEM((1, 64), jnp.float32),
            pltpu.VMEM((1, 64), jnp.float32),
        ],
    )(h_i, acc2[:, :N, :], W_na[:64], W_na[64:80], hg, W_na[80:96],
      b_na.reshape(1, 64), W_l, W_r, b_sc.reshape(1, 64))

    return out
